# baseline (speedup 1.0000x reference)
import functools

import jax
import jax.numpy as jnp
from jax import lax
from jax.experimental import pallas as pl
from jax.experimental.pallas import tpu as pltpu
from jax.experimental.pallas import tpu_sc as plsc

NC = 2
NS = 16
NW = NC * NS
CH = 72
DEPTH = 3


def _sc_gather_body(n_ch, idxl_hbm, idxr_hbm, h_hbm, c_hbm,
                    hl_out, hr_out, cl_out, cr_out,
                    idxl_v, idxr_v, *bufs_and_sems):
    wid = lax.axis_index("s") * NC + lax.axis_index("c")
    base = wid * (n_ch * CH)
    pltpu.sync_copy(idxl_hbm.at[pl.ds(base, n_ch * CH)], idxl_v)
    pltpu.sync_copy(idxr_hbm.at[pl.ds(base, n_ch * CH)], idxr_v)
    slots = [bufs_and_sems[5 * d:5 * d + 5] for d in range(DEPTH)]
    g_desc = [None] * DEPTH
    w_desc = [None] * DEPTH

    def gather(j):
        bhl, bhr, bcl, bcr, sem = slots[j % DEPTH]
        il = idxl_v.at[pl.ds(j * CH, CH)]
        ir = idxr_v.at[pl.ds(j * CH, CH)]
        g_desc[j % DEPTH] = [
            pltpu.async_copy(h_hbm.at[il], bhl, sem),
            pltpu.async_copy(h_hbm.at[ir], bhr, sem),
            pltpu.async_copy(c_hbm.at[il], bcl, sem),
            pltpu.async_copy(c_hbm.at[ir], bcr, sem),
        ]

    def issue_writeback(j):
        bhl, bhr, bcl, bcr, sem = slots[j % DEPTH]
        off = base + j * CH
        w_desc[j % DEPTH] = [
            pltpu.async_copy(bhl, hl_out.at[pl.ds(off, CH)], sem),
            pltpu.async_copy(bhr, hr_out.at[pl.ds(off, CH)], sem),
            pltpu.async_copy(bcl, cl_out.at[pl.ds(off, CH)], sem),
            pltpu.async_copy(bcr, cr_out.at[pl.ds(off, CH)], sem),
        ]

    def wait_w(j):
        for d in w_desc[j % DEPTH]:
            d.wait()

    gather(0)
    if n_ch > 1:
        gather(1)
    for j in range(n_ch):
        if j + 2 < n_ch:
            if j >= 1:
                wait_w(j - 1)
            gather(j + 2)
        for d in g_desc[j % DEPTH]:
            d.wait()
        issue_writeback(j)
    for k in range(max(0, n_ch - DEPTH), n_ch):
        wait_w(k)


def _sc_gather(idxl, idxr, h, c, n_ch):
    Hs = h.shape[1]
    P_pad = idxl.shape[0]
    out = jax.ShapeDtypeStruct((P_pad, Hs), jnp.float32)
    mesh = plsc.VectorSubcoreMesh(core_axis_name="c", subcore_axis_name="s")
    idxs = pltpu.VMEM((n_ch * CH,), jnp.int32)
    row = pltpu.VMEM((CH, Hs), jnp.float32)
    scratch = [idxs, idxs]
    for _ in range(DEPTH):
        scratch.extend([row, row, row, row, pltpu.SemaphoreType.DMA])
    fn = pl.kernel(
        functools.partial(_sc_gather_body, n_ch),
        out_type=(out, out, out, out),
        mesh=mesh,
        scratch_types=scratch,
    )
    return fn(idxl, idxr, h, c)


def _tc_body(Hs, hl_ref, hr_ref, cl_ref, cr_ref, W_ref, b_ref, ho_ref, co_ref):
    hl = hl_ref[...]
    hr = hr_ref[...]
    z = (jnp.dot(hl, W_ref[:Hs, :], preferred_element_type=jnp.float32)
         + jnp.dot(hr, W_ref[Hs:, :], preferred_element_type=jnp.float32)
         + b_ref[...])
    f = jax.nn.sigmoid(z[:, :2 * Hs])
    c_sum = f[:, :Hs] * cl_ref[...] + f[:, Hs:] * cr_ref[...]
    i = jax.nn.sigmoid(z[:, 2 * Hs:3 * Hs])
    o = jax.nn.sigmoid(z[:, 3 * Hs:4 * Hs])
    u = jnp.tanh(z[:, 4 * Hs:])
    c_out = i * u + c_sum
    co_ref[...] = c_out
    ho_ref[...] = o * jnp.tanh(c_out)


def _tc_dense(hl, hr, cl, cr, W, b, P, bp):
    Hs = hl.shape[1]
    grid = P // bp
    blk = pl.BlockSpec((bp, Hs), lambda i: (i, 0))
    wblk = pl.BlockSpec(W.shape, lambda i: (0, 0))
    bblk = pl.BlockSpec(b.shape, lambda i: (0, 0))
    out = jax.ShapeDtypeStruct((P, Hs), jnp.float32)
    return pl.pallas_call(
        functools.partial(_tc_body, Hs),
        grid=(grid,),
        in_specs=[blk, blk, blk, blk, wblk, bblk],
        out_specs=[blk, blk],
        out_shape=(out, out),
    )(hl, hr, cl, cr, W, b)


def kernel(h, c, child_idx, U_f, b_f, U_iou, b_iou):
    P = child_idx.shape[0]
    idx = child_idx.astype(jnp.int32)
    step = NW * CH
    P_pad = ((P + step - 1) // step) * step
    n_ch = P_pad // step
    pad = P_pad - P
    idxl = jnp.pad(idx[:, 0], (0, pad))
    idxr = jnp.pad(idx[:, 1], (0, pad))

    hl, hr, cl, cr = _sc_gather(idxl, idxr, h, c, n_ch)

    W = jnp.concatenate([U_f, U_iou], axis=1)
    b = jnp.concatenate([b_f, b_iou.reshape(-1)]).reshape(1, -1)
    bp = 2000
    while P % bp != 0:
        bp //= 2
    return _tc_dense(hl, hr, cl, cr, W, b, P, bp)


# speedup vs baseline: 1.3115x; 1.3115x over previous
import functools

import jax
import jax.numpy as jnp
from jax import lax
from jax.experimental import pallas as pl
from jax.experimental.pallas import tpu as pltpu
from jax.experimental.pallas import tpu_sc as plsc

NC = 2
NS = 16
NW = NC * NS
CH = 112
DEPTH = 2


def _sc_gather_body(n_ch, idxl_hbm, idxr_hbm, h_hbm, c_hbm,
                    hl_out, hr_out, cl_out, cr_out,
                    idxl_v, idxr_v, *bufs_and_sems):
    wid = lax.axis_index("s") * NC + lax.axis_index("c")
    base = wid * (n_ch * CH)
    pltpu.sync_copy(idxl_hbm.at[pl.ds(base, n_ch * CH)], idxl_v)
    pltpu.sync_copy(idxr_hbm.at[pl.ds(base, n_ch * CH)], idxr_v)
    slots = [bufs_and_sems[5 * d:5 * d + 5] for d in range(DEPTH)]
    g_desc = [None] * DEPTH
    w_desc = [None] * DEPTH

    def gather(j):
        bhl, bhr, bcl, bcr, sem = slots[j % DEPTH]
        il = idxl_v.at[pl.ds(j * CH, CH)]
        ir = idxr_v.at[pl.ds(j * CH, CH)]
        g_desc[j % DEPTH] = [
            pltpu.async_copy(h_hbm.at[il], bhl, sem),
            pltpu.async_copy(h_hbm.at[ir], bhr, sem),
            pltpu.async_copy(c_hbm.at[il], bcl, sem),
            pltpu.async_copy(c_hbm.at[ir], bcr, sem),
        ]

    def issue_writeback(j):
        bhl, bhr, bcl, bcr, sem = slots[j % DEPTH]
        off = base + j * CH
        w_desc[j % DEPTH] = [
            pltpu.async_copy(bhl, hl_out.at[pl.ds(off, CH)], sem),
            pltpu.async_copy(bhr, hr_out.at[pl.ds(off, CH)], sem),
            pltpu.async_copy(bcl, cl_out.at[pl.ds(off, CH)], sem),
            pltpu.async_copy(bcr, cr_out.at[pl.ds(off, CH)], sem),
        ]

    def wait_w(j):
        for d in w_desc[j % DEPTH]:
            d.wait()

    gather(0)
    for j in range(n_ch):
        if j + 1 < n_ch:
            if j >= 1:
                wait_w(j - 1)
            gather(j + 1)
        for d in g_desc[j % DEPTH]:
            d.wait()
        issue_writeback(j)
    for k in range(max(0, n_ch - DEPTH), n_ch):
        wait_w(k)


def _sc_gather(idxl, idxr, h, c, n_ch):
    Hs = h.shape[1]
    P_pad = idxl.shape[0]
    out = jax.ShapeDtypeStruct((P_pad, Hs), jnp.float32)
    mesh = plsc.VectorSubcoreMesh(core_axis_name="c", subcore_axis_name="s")
    idxs = pltpu.VMEM((n_ch * CH,), jnp.int32)
    row = pltpu.VMEM((CH, Hs), jnp.float32)
    scratch = [idxs, idxs]
    for _ in range(DEPTH):
        scratch.extend([row, row, row, row, pltpu.SemaphoreType.DMA])
    fn = pl.kernel(
        functools.partial(_sc_gather_body, n_ch),
        out_type=(out, out, out, out),
        mesh=mesh,
        scratch_types=scratch,
    )
    return fn(idxl, idxr, h, c)


def _tc_body(Hs, hl_ref, hr_ref, cl_ref, cr_ref, W_ref, b_ref, ho_ref, co_ref):
    hl = hl_ref[...]
    hr = hr_ref[...]
    z = (jnp.dot(hl, W_ref[:Hs, :], preferred_element_type=jnp.float32)
         + jnp.dot(hr, W_ref[Hs:, :], preferred_element_type=jnp.float32)
         + b_ref[...])
    f = jax.nn.sigmoid(z[:, :2 * Hs])
    c_sum = f[:, :Hs] * cl_ref[...] + f[:, Hs:] * cr_ref[...]
    i = jax.nn.sigmoid(z[:, 2 * Hs:3 * Hs])
    o = jax.nn.sigmoid(z[:, 3 * Hs:4 * Hs])
    u = jnp.tanh(z[:, 4 * Hs:])
    c_out = i * u + c_sum
    co_ref[...] = c_out
    ho_ref[...] = o * jnp.tanh(c_out)


def _tc_dense(hl, hr, cl, cr, W, b, P, bp):
    Hs = hl.shape[1]
    grid = P // bp
    blk = pl.BlockSpec((bp, Hs), lambda i: (i, 0))
    wblk = pl.BlockSpec(W.shape, lambda i: (0, 0))
    bblk = pl.BlockSpec(b.shape, lambda i: (0, 0))
    out = jax.ShapeDtypeStruct((P, Hs), jnp.float32)
    return pl.pallas_call(
        functools.partial(_tc_body, Hs),
        grid=(grid,),
        in_specs=[blk, blk, blk, blk, wblk, bblk],
        out_specs=[blk, blk],
        out_shape=(out, out),
    )(hl, hr, cl, cr, W, b)


def kernel(h, c, child_idx, U_f, b_f, U_iou, b_iou):
    P = child_idx.shape[0]
    idx = child_idx.astype(jnp.int32)
    step = NW * CH
    P_pad = ((P + step - 1) // step) * step
    n_ch = P_pad // step
    pad = P_pad - P
    idxl = jnp.pad(idx[:, 0], (0, pad))
    idxr = jnp.pad(idx[:, 1], (0, pad))

    hl, hr, cl, cr = _sc_gather(idxl, idxr, h, c, n_ch)

    W = jnp.concatenate([U_f, U_iou], axis=1)
    b = jnp.concatenate([b_f, b_iou.reshape(-1)]).reshape(1, -1)
    bp = 2000
    while P % bp != 0:
        bp //= 2
    return _tc_dense(hl, hr, cl, cr, W, b, P, bp)
